# Initial kernel scaffold; baseline (speedup 1.0000x reference)
import jax
import jax.numpy as jnp
from jax.experimental import pallas as pl


def kernel(encoded_identifiers, symbols_identifier_indices, encodings_of_symbols_occurrences, symbols_indices_of_symbols_occurrences, W_comb):
    raise NotImplementedError("write your pallas kernel here")


# speedup vs baseline: 3.4078x; 3.4078x over previous
import jax
import jax.numpy as jnp
from jax import lax
from jax.experimental import pallas as pl
from jax.experimental.pallas import tpu as pltpu
from jax.experimental.pallas import tpu_sc as plsc

NR_IDENTIFIERS = 20000
NR_SYMBOLS = 10000
NR_OCCURRENCES = 320000
D = 128

NC = 2
NS = 16
NW = NC * NS

SYM_PAD = 10240
OCC_PER_TILE = NR_OCCURRENCES // NW
KROWS = 80
NCHUNK = OCC_PER_TILE // KROWS
GPT = 384
GATHER_PAD = NW * GPT

ROWS_PER_TILE = SYM_PAD // NS
LANES = 16


def _sc_body(occ_hbm, occidx_hbm, z128_hbm, z1d_hbm,
             sums_out, cnts_out,
             rows_v, idx_v, cnt_v, sums_sh):
    c = lax.axis_index("c")
    s = lax.axis_index("s")
    w = c * NS + s
    zlo = s * ROWS_PER_TILE
    olo = c * SYM_PAD + zlo

    pltpu.sync_copy(z128_hbm, rows_v)

    def zinit(k, carry):
        pltpu.sync_copy(rows_v, sums_sh.at[pl.ds(zlo + k * KROWS, KROWS)])
        return carry

    lax.fori_loop(0, ROWS_PER_TILE // KROWS, zinit, 0)
    pltpu.sync_copy(z1d_hbm, cnt_v)

    plsc.subcore_barrier()

    base = w * OCC_PER_TILE
    ones16 = jnp.ones((LANES,), jnp.float32)

    def chunk(j, carry):
        off = pl.multiple_of(base + j * KROWS, 8)
        pltpu.sync_copy(occidx_hbm.at[pl.ds(off, KROWS)], idx_v)
        pltpu.sync_copy(occ_hbm.at[pl.ds(off, KROWS)], rows_v)
        pltpu.sync_copy(rows_v, sums_sh.at[idx_v], add=True)
        for t in range(KROWS // LANES):
            iv = idx_v[pl.ds(t * LANES, LANES)]
            plsc.addupdate_scatter(cnt_v, [iv], ones16)
        return carry

    lax.fori_loop(0, NCHUNK, chunk, 0)

    plsc.subcore_barrier()

    def wout(k, carry):
        pltpu.sync_copy(sums_sh.at[pl.ds(zlo + k * KROWS, KROWS)], rows_v)
        pltpu.sync_copy(rows_v, sums_out.at[pl.ds(olo + k * KROWS, KROWS)])
        return carry

    lax.fori_loop(0, ROWS_PER_TILE // KROWS, wout, 0)
    pltpu.sync_copy(cnt_v, cnts_out.at[w])


def _gather_body(enc_hbm, gidx_hbm, gath_out, gidx_v, grows_v):
    c = lax.axis_index("c")
    s = lax.axis_index("s")
    w = c * NS + s
    pltpu.sync_copy(gidx_hbm.at[w], gidx_v)
    for j in range(GPT // 128):
        pltpu.sync_copy(enc_hbm.at[gidx_v.at[j]],
                        grows_v.at[pl.ds(j * 128, 128)])
    pltpu.sync_copy(grows_v, gath_out.at[pl.ds(w * GPT, GPT)])


def _sc_mesh():
    return plsc.VectorSubcoreMesh(core_axis_name="c", subcore_axis_name="s",
                                  num_cores=NC, num_subcores=NS)


@jax.jit
def _sc_stage(occ, occ_idx, z128, z1d):
    return pl.kernel(
        _sc_body,
        out_type=[
            jax.ShapeDtypeStruct((NC * SYM_PAD, D), jnp.float32),
            jax.ShapeDtypeStruct((NW, SYM_PAD), jnp.float32),
        ],
        mesh=_sc_mesh(),
        scratch_types=[
            pltpu.VMEM((KROWS, D), jnp.float32),
            pltpu.VMEM((KROWS,), jnp.int32),
            pltpu.VMEM((SYM_PAD,), jnp.float32),
            pltpu.VMEM_SHARED((SYM_PAD, D), jnp.float32),
        ],
        compiler_params=pltpu.CompilerParams(needs_layout_passes=False),
    )(occ, occ_idx, z128, z1d)


@jax.jit
def _gather_stage(enc, gidx):
    return pl.kernel(
        _gather_body,
        out_type=[jax.ShapeDtypeStruct((GATHER_PAD, D), jnp.float32)],
        mesh=_sc_mesh(),
        scratch_types=[
            pltpu.VMEM((GPT // 128, 128), jnp.int32),
            pltpu.VMEM((GPT, D), jnp.float32),
        ],
    )(enc, gidx)


BT = 1024


def _tc_body(g_ref, s0_ref, s1_ref, c_ref, w1_ref, w2_ref, o_ref):
    cnt = jnp.maximum(jnp.sum(c_ref[...], axis=0), 1.0)
    acc = jnp.dot(g_ref[...], w1_ref[...], preferred_element_type=jnp.float32)
    acc2 = jnp.dot(s0_ref[...] + s1_ref[...], w2_ref[...],
                   preferred_element_type=jnp.float32)
    o_ref[...] = jnp.maximum(acc + acc2 / cnt[:, None], 0.0)


@jax.jit
def _tc_stage(g, s0, s1, cnts, w1t, w2t):
    grid = (SYM_PAD // BT,)
    return pl.pallas_call(
        _tc_body,
        grid=grid,
        in_specs=[
            pl.BlockSpec((BT, D), lambda i: (i, 0)),
            pl.BlockSpec((BT, D), lambda i: (i, 0)),
            pl.BlockSpec((BT, D), lambda i: (i, 0)),
            pl.BlockSpec((NW, BT), lambda i: (0, i)),
            pl.BlockSpec((D, D), lambda i: (0, 0)),
            pl.BlockSpec((D, D), lambda i: (0, 0)),
        ],
        out_specs=pl.BlockSpec((BT, D), lambda i: (i, 0)),
        out_shape=jax.ShapeDtypeStruct((SYM_PAD, D), jnp.float32),
    )(g, s0, s1, cnts, w1t, w2t)


def kernel(encoded_identifiers, symbols_identifier_indices,
           encodings_of_symbols_occurrences,
           symbols_indices_of_symbols_occurrences, W_comb):
    occ = encodings_of_symbols_occurrences
    occ_idx = symbols_indices_of_symbols_occurrences.astype(jnp.int32)

    gidx = jnp.zeros((GATHER_PAD,), jnp.int32)
    gidx = gidx.at[:NR_SYMBOLS].set(
        symbols_identifier_indices.astype(jnp.int32))
    gidx = gidx.reshape(NW, GPT // 128, 128)

    z128 = jnp.zeros((KROWS, D), jnp.float32)
    z1d = jnp.zeros((SYM_PAD,), jnp.float32)

    sums_f, cnts_p = _sc_stage(occ, occ_idx, z128, z1d)
    sums_p = sums_f.reshape(NC, SYM_PAD, D)
    gath = _gather_stage(encoded_identifiers, gidx)
    if isinstance(gath, (list, tuple)):
        gath = gath[0]

    w1t = W_comb[:, :D].T
    w2t = W_comb[:, D:].T
    out = _tc_stage(gath[:SYM_PAD], sums_p[0], sums_p[1], cnts_p, w1t, w2t)
    return out[:NR_SYMBOLS]


# speedup vs baseline: 5.2014x; 1.5263x over previous
import jax
import jax.numpy as jnp
from jax import lax
from jax.experimental import pallas as pl
from jax.experimental.pallas import tpu as pltpu
from jax.experimental.pallas import tpu_sc as plsc

NR_IDENTIFIERS = 20000
NR_SYMBOLS = 10000
NR_OCCURRENCES = 320000
D = 128

NC = 2
NS = 16
NW = NC * NS

SYM_PAD = 10240
OCC_PER_TILE = NR_OCCURRENCES // NW
KROWS = 80
NCHUNK = OCC_PER_TILE // KROWS
GPT = 384
GATHER_PAD = NW * GPT

ROWS_PER_TILE = SYM_PAD // NS
LANES = 16


def _sc_body(occ_hbm, occidx_hbm, z128_hbm, z1d_hbm,
             sums_out, cnts_out,
             rows_v, idx_v, rows_b, idx_b, cnt_v, sums_sh,
             sem_ra, sem_ia, sem_rb, sem_ib):
    c = lax.axis_index("c")
    s = lax.axis_index("s")
    w = c * NS + s
    zlo = s * ROWS_PER_TILE
    olo = c * SYM_PAD + zlo

    pltpu.sync_copy(z128_hbm, rows_v)

    def zinit(k, carry):
        pltpu.sync_copy(rows_v, sums_sh.at[pl.ds(zlo + k * KROWS, KROWS)])
        return carry

    lax.fori_loop(0, ROWS_PER_TILE // KROWS, zinit, 0)
    pltpu.sync_copy(z1d_hbm, cnt_v)

    plsc.subcore_barrier()

    base = w * OCC_PER_TILE
    ones16 = jnp.ones((LANES,), jnp.float32)

    def start_load(off, rows_buf, idx_buf, sem_r, sem_i):
        pltpu.async_copy(occ_hbm.at[pl.ds(off, KROWS)], rows_buf, sem_r)
        pltpu.async_copy(occidx_hbm.at[pl.ds(off, KROWS)], idx_buf, sem_i)

    def wait_load(off, rows_buf, idx_buf, sem_r, sem_i):
        pltpu.make_async_copy(
            occ_hbm.at[pl.ds(off, KROWS)], rows_buf, sem_r).wait()
        pltpu.make_async_copy(
            occidx_hbm.at[pl.ds(off, KROWS)], idx_buf, sem_i).wait()

    def process(rows_buf, idx_buf):
        pltpu.sync_copy(rows_buf, sums_sh.at[idx_buf], add=True)
        for t in range(KROWS // LANES):
            iv = idx_buf[pl.ds(t * LANES, LANES)]
            plsc.addupdate_scatter(cnt_v, [iv], ones16)

    start_load(pl.multiple_of(base, 8), rows_v, idx_v, sem_ra, sem_ia)

    def pair(k, carry):
        j = 2 * k
        off_a = pl.multiple_of(base + j * KROWS, 8)
        off_b = pl.multiple_of(base + (j + 1) * KROWS, 8)
        off_n = pl.multiple_of(
            base + jnp.minimum(j + 2, NCHUNK - 1) * KROWS, 8)
        start_load(off_b, rows_b, idx_b, sem_rb, sem_ib)
        wait_load(off_a, rows_v, idx_v, sem_ra, sem_ia)
        process(rows_v, idx_v)
        start_load(off_n, rows_v, idx_v, sem_ra, sem_ia)
        wait_load(off_b, rows_b, idx_b, sem_rb, sem_ib)
        process(rows_b, idx_b)
        return carry

    lax.fori_loop(0, NCHUNK // 2, pair, 0)
    off_t = pl.multiple_of(base + (NCHUNK - 1) * KROWS, 8)
    wait_load(off_t, rows_v, idx_v, sem_ra, sem_ia)
    process(rows_v, idx_v)

    plsc.subcore_barrier()

    def wout(k, carry):
        pltpu.sync_copy(sums_sh.at[pl.ds(zlo + k * KROWS, KROWS)], rows_v)
        pltpu.sync_copy(rows_v, sums_out.at[pl.ds(olo + k * KROWS, KROWS)])
        return carry

    lax.fori_loop(0, ROWS_PER_TILE // KROWS, wout, 0)
    pltpu.sync_copy(cnt_v, cnts_out.at[w])


def _gather_body(enc_hbm, gidx_hbm, gath_out, gidx_v, grows_v):
    c = lax.axis_index("c")
    s = lax.axis_index("s")
    w = c * NS + s
    pltpu.sync_copy(gidx_hbm.at[w], gidx_v)
    for j in range(GPT // 128):
        pltpu.sync_copy(enc_hbm.at[gidx_v.at[j]],
                        grows_v.at[pl.ds(j * 128, 128)])
    pltpu.sync_copy(grows_v, gath_out.at[pl.ds(w * GPT, GPT)])


def _sc_mesh():
    return plsc.VectorSubcoreMesh(core_axis_name="c", subcore_axis_name="s",
                                  num_cores=NC, num_subcores=NS)


@jax.jit
def _sc_stage(occ, occ_idx, z128, z1d):
    return pl.kernel(
        _sc_body,
        out_type=[
            jax.ShapeDtypeStruct((NC * SYM_PAD, D), jnp.float32),
            jax.ShapeDtypeStruct((NW, SYM_PAD), jnp.float32),
        ],
        mesh=_sc_mesh(),
        scratch_types=[
            pltpu.VMEM((KROWS, D), jnp.float32),
            pltpu.VMEM((KROWS,), jnp.int32),
            pltpu.VMEM((KROWS, D), jnp.float32),
            pltpu.VMEM((KROWS,), jnp.int32),
            pltpu.VMEM((SYM_PAD,), jnp.float32),
            pltpu.VMEM_SHARED((SYM_PAD, D), jnp.float32),
            pltpu.SemaphoreType.DMA,
            pltpu.SemaphoreType.DMA,
            pltpu.SemaphoreType.DMA,
            pltpu.SemaphoreType.DMA,
        ],
        compiler_params=pltpu.CompilerParams(needs_layout_passes=False),
    )(occ, occ_idx, z128, z1d)


@jax.jit
def _gather_stage(enc, gidx):
    return pl.kernel(
        _gather_body,
        out_type=[jax.ShapeDtypeStruct((GATHER_PAD, D), jnp.float32)],
        mesh=_sc_mesh(),
        scratch_types=[
            pltpu.VMEM((GPT // 128, 128), jnp.int32),
            pltpu.VMEM((GPT, D), jnp.float32),
        ],
    )(enc, gidx)


BT = 1024


def _tc_body(g_ref, s0_ref, s1_ref, c_ref, w1_ref, w2_ref, o_ref):
    cnt = jnp.maximum(jnp.sum(c_ref[...], axis=0), 1.0)
    acc = jnp.dot(g_ref[...], w1_ref[...], preferred_element_type=jnp.float32)
    acc2 = jnp.dot(s0_ref[...] + s1_ref[...], w2_ref[...],
                   preferred_element_type=jnp.float32)
    o_ref[...] = jnp.maximum(acc + acc2 / cnt[:, None], 0.0)


@jax.jit
def _tc_stage(g, s0, s1, cnts, w1t, w2t):
    grid = (SYM_PAD // BT,)
    return pl.pallas_call(
        _tc_body,
        grid=grid,
        in_specs=[
            pl.BlockSpec((BT, D), lambda i: (i, 0)),
            pl.BlockSpec((BT, D), lambda i: (i, 0)),
            pl.BlockSpec((BT, D), lambda i: (i, 0)),
            pl.BlockSpec((NW, BT), lambda i: (0, i)),
            pl.BlockSpec((D, D), lambda i: (0, 0)),
            pl.BlockSpec((D, D), lambda i: (0, 0)),
        ],
        out_specs=pl.BlockSpec((BT, D), lambda i: (i, 0)),
        out_shape=jax.ShapeDtypeStruct((SYM_PAD, D), jnp.float32),
    )(g, s0, s1, cnts, w1t, w2t)


def kernel(encoded_identifiers, symbols_identifier_indices,
           encodings_of_symbols_occurrences,
           symbols_indices_of_symbols_occurrences, W_comb):
    occ = encodings_of_symbols_occurrences
    occ_idx = symbols_indices_of_symbols_occurrences.astype(jnp.int32)

    gidx = jnp.zeros((GATHER_PAD,), jnp.int32)
    gidx = gidx.at[:NR_SYMBOLS].set(
        symbols_identifier_indices.astype(jnp.int32))
    gidx = gidx.reshape(NW, GPT // 128, 128)

    z128 = jnp.zeros((KROWS, D), jnp.float32)
    z1d = jnp.zeros((SYM_PAD,), jnp.float32)

    sums_f, cnts_p = _sc_stage(occ, occ_idx, z128, z1d)
    sums_p = sums_f.reshape(NC, SYM_PAD, D)
    gath = _gather_stage(encoded_identifiers, gidx)
    if isinstance(gath, (list, tuple)):
        gath = gath[0]

    w1t = W_comb[:, :D].T
    w2t = W_comb[:, D:].T
    out = _tc_stage(gath[:SYM_PAD], sums_p[0], sums_p[1], cnts_p, w1t, w2t)
    return out[:NR_SYMBOLS]


# speedup vs baseline: 5.3100x; 1.0209x over previous
import jax
import jax.numpy as jnp
from jax import lax
from jax.experimental import pallas as pl
from jax.experimental.pallas import tpu as pltpu
from jax.experimental.pallas import tpu_sc as plsc

NR_IDENTIFIERS = 20000
NR_SYMBOLS = 10000
NR_OCCURRENCES = 320000
D = 128

NC = 2
NS = 16
NW = NC * NS

SYM_PAD = 10240
OCC_PER_TILE = NR_OCCURRENCES // NW
KROWS = 80
NCHUNK = OCC_PER_TILE // KROWS
GPT = 384
GATHER_PAD = NW * GPT

ROWS_PER_TILE = SYM_PAD // NS
LANES = 16


def _sc_body(occ_hbm, occidx_hbm, z128_hbm, z1d_hbm,
             sums_out, cnts_out,
             rows_v, idx_v, rows_b, idx_b, cnt_v, sums_sh,
             sem_ra, sem_ia, sem_rb, sem_ib, sem_sa, sem_sb):
    c = lax.axis_index("c")
    s = lax.axis_index("s")
    w = c * NS + s
    zlo = s * ROWS_PER_TILE
    olo = c * SYM_PAD + zlo

    pltpu.sync_copy(z128_hbm, rows_v)

    def zinit(k, carry):
        pltpu.sync_copy(rows_v, sums_sh.at[pl.ds(zlo + k * KROWS, KROWS)])
        return carry

    lax.fori_loop(0, ROWS_PER_TILE // KROWS, zinit, 0)
    pltpu.sync_copy(z1d_hbm, cnt_v)

    plsc.subcore_barrier()

    base = w * OCC_PER_TILE
    ones16 = jnp.ones((LANES,), jnp.float32)

    def start_load(off, rows_buf, idx_buf, sem_r, sem_i):
        pltpu.async_copy(occ_hbm.at[pl.ds(off, KROWS)], rows_buf, sem_r)
        pltpu.async_copy(occidx_hbm.at[pl.ds(off, KROWS)], idx_buf, sem_i)

    def wait_load(off, rows_buf, idx_buf, sem_r, sem_i):
        pltpu.make_async_copy(
            occ_hbm.at[pl.ds(off, KROWS)], rows_buf, sem_r).wait()
        pltpu.make_async_copy(
            occidx_hbm.at[pl.ds(off, KROWS)], idx_buf, sem_i).wait()

    def process(rows_buf, idx_buf, sem_s):
        d = pltpu.async_copy(rows_buf, sums_sh.at[idx_buf], sem_s, add=True)
        for t in range(KROWS // LANES):
            iv = idx_buf[pl.ds(t * LANES, LANES)]
            plsc.addupdate_scatter(cnt_v, [iv], ones16)
        d.wait()

    start_load(pl.multiple_of(base, 8), rows_v, idx_v, sem_ra, sem_ia)

    def pair(k, carry):
        j = 2 * k
        off_a = pl.multiple_of(base + j * KROWS, 8)
        off_b = pl.multiple_of(base + (j + 1) * KROWS, 8)
        off_n = pl.multiple_of(
            base + jnp.minimum(j + 2, NCHUNK - 1) * KROWS, 8)
        start_load(off_b, rows_b, idx_b, sem_rb, sem_ib)
        wait_load(off_a, rows_v, idx_v, sem_ra, sem_ia)
        process(rows_v, idx_v, sem_sa)
        start_load(off_n, rows_v, idx_v, sem_ra, sem_ia)
        wait_load(off_b, rows_b, idx_b, sem_rb, sem_ib)
        process(rows_b, idx_b, sem_sb)
        return carry

    lax.fori_loop(0, NCHUNK // 2, pair, 0)
    off_t = pl.multiple_of(base + (NCHUNK - 1) * KROWS, 8)
    wait_load(off_t, rows_v, idx_v, sem_ra, sem_ia)
    process(rows_v, idx_v, sem_sa)

    plsc.subcore_barrier()

    def wout(k, carry):
        pltpu.sync_copy(sums_sh.at[pl.ds(zlo + k * KROWS, KROWS)], rows_v)
        pltpu.sync_copy(rows_v, sums_out.at[pl.ds(olo + k * KROWS, KROWS)])
        return carry

    lax.fori_loop(0, ROWS_PER_TILE // KROWS, wout, 0)
    pltpu.sync_copy(cnt_v, cnts_out.at[w])


def _gather_body(enc_hbm, gidx_hbm, gath_out, gidx_v, grows_v):
    c = lax.axis_index("c")
    s = lax.axis_index("s")
    w = c * NS + s
    pltpu.sync_copy(gidx_hbm.at[w], gidx_v)
    for j in range(GPT // 128):
        pltpu.sync_copy(enc_hbm.at[gidx_v.at[j]],
                        grows_v.at[pl.ds(j * 128, 128)])
    pltpu.sync_copy(grows_v, gath_out.at[pl.ds(w * GPT, GPT)])


def _sc_mesh():
    return plsc.VectorSubcoreMesh(core_axis_name="c", subcore_axis_name="s",
                                  num_cores=NC, num_subcores=NS)


@jax.jit
def _sc_stage(occ, occ_idx, z128, z1d):
    return pl.kernel(
        _sc_body,
        out_type=[
            jax.ShapeDtypeStruct((NC * SYM_PAD, D), jnp.float32),
            jax.ShapeDtypeStruct((NW, SYM_PAD), jnp.float32),
        ],
        mesh=_sc_mesh(),
        scratch_types=[
            pltpu.VMEM((KROWS, D), jnp.float32),
            pltpu.VMEM((KROWS,), jnp.int32),
            pltpu.VMEM((KROWS, D), jnp.float32),
            pltpu.VMEM((KROWS,), jnp.int32),
            pltpu.VMEM((SYM_PAD,), jnp.float32),
            pltpu.VMEM_SHARED((SYM_PAD, D), jnp.float32),
            pltpu.SemaphoreType.DMA,
            pltpu.SemaphoreType.DMA,
            pltpu.SemaphoreType.DMA,
            pltpu.SemaphoreType.DMA,
            pltpu.SemaphoreType.DMA,
            pltpu.SemaphoreType.DMA,
        ],
        compiler_params=pltpu.CompilerParams(needs_layout_passes=False),
    )(occ, occ_idx, z128, z1d)


@jax.jit
def _gather_stage(enc, gidx):
    return pl.kernel(
        _gather_body,
        out_type=[jax.ShapeDtypeStruct((GATHER_PAD, D), jnp.float32)],
        mesh=_sc_mesh(),
        scratch_types=[
            pltpu.VMEM((GPT // 128, 128), jnp.int32),
            pltpu.VMEM((GPT, D), jnp.float32),
        ],
    )(enc, gidx)


BT = 1024


def _tc_body(g_ref, s0_ref, s1_ref, c_ref, w1_ref, w2_ref, o_ref):
    cnt = jnp.maximum(jnp.sum(c_ref[...], axis=0), 1.0)
    acc = jnp.dot(g_ref[...], w1_ref[...], preferred_element_type=jnp.float32)
    acc2 = jnp.dot(s0_ref[...] + s1_ref[...], w2_ref[...],
                   preferred_element_type=jnp.float32)
    o_ref[...] = jnp.maximum(acc + acc2 / cnt[:, None], 0.0)


@jax.jit
def _tc_stage(g, s0, s1, cnts, w1t, w2t):
    grid = (SYM_PAD // BT,)
    return pl.pallas_call(
        _tc_body,
        grid=grid,
        in_specs=[
            pl.BlockSpec((BT, D), lambda i: (i, 0)),
            pl.BlockSpec((BT, D), lambda i: (i, 0)),
            pl.BlockSpec((BT, D), lambda i: (i, 0)),
            pl.BlockSpec((NW, BT), lambda i: (0, i)),
            pl.BlockSpec((D, D), lambda i: (0, 0)),
            pl.BlockSpec((D, D), lambda i: (0, 0)),
        ],
        out_specs=pl.BlockSpec((BT, D), lambda i: (i, 0)),
        out_shape=jax.ShapeDtypeStruct((SYM_PAD, D), jnp.float32),
    )(g, s0, s1, cnts, w1t, w2t)


def kernel(encoded_identifiers, symbols_identifier_indices,
           encodings_of_symbols_occurrences,
           symbols_indices_of_symbols_occurrences, W_comb):
    occ = encodings_of_symbols_occurrences
    occ_idx = symbols_indices_of_symbols_occurrences.astype(jnp.int32)

    gidx = jnp.zeros((GATHER_PAD,), jnp.int32)
    gidx = gidx.at[:NR_SYMBOLS].set(
        symbols_identifier_indices.astype(jnp.int32))
    gidx = gidx.reshape(NW, GPT // 128, 128)

    z128 = jnp.zeros((KROWS, D), jnp.float32)
    z1d = jnp.zeros((SYM_PAD,), jnp.float32)

    sums_f, cnts_p = _sc_stage(occ, occ_idx, z128, z1d)
    sums_p = sums_f.reshape(NC, SYM_PAD, D)
    gath = _gather_stage(encoded_identifiers, gidx)
    if isinstance(gath, (list, tuple)):
        gath = gath[0]

    w1t = W_comb[:, :D].T
    w2t = W_comb[:, D:].T
    out = _tc_stage(gath[:SYM_PAD], sums_p[0], sums_p[1], cnts_p, w1t, w2t)
    return out[:NR_SYMBOLS]
